# baseline (device time: 8941 ns/iter reference)
import jax
import jax.numpy as jnp
from jax import lax
from jax.experimental import pallas as pl
from jax.experimental.pallas import tpu as pltpu


def kernel(x):
    m_per, n_per = x.shape
    bm = 256
    nblk = m_per // bm
    rows_per_blk = bm // 128

    def body(x_ref, out_ref, acc_ref, peer_ref, send_sem, recv_sem):
        i = pl.program_id(0)
        my_x = lax.axis_index("x")
        my_y = lax.axis_index("y")
        peer = (my_x, 1 - my_y)
        barrier_sem = pltpu.get_barrier_semaphore()

        @pl.when(i == 0)
        def _():
            pl.semaphore_signal(
                barrier_sem, inc=1, device_id=peer,
                device_id_type=pl.DeviceIdType.MESH,
            )

        s1 = x_ref[:, 0:128]
        for j in range(1, n_per // 128):
            s1 = jnp.maximum(s1, x_ref[:, j * 128:(j + 1) * 128])
        part = jnp.max(s1, axis=1)
        acc_ref[pl.ds(i * rows_per_blk, rows_per_blk), :] = jnp.reshape(
            part, (rows_per_blk, 128)
        )

        half = (nblk // 2) * rows_per_blk

        def half_rdma(lo, hi, slot):
            return pltpu.make_async_remote_copy(
                src_ref=acc_ref.at[pl.ds(lo, hi - lo)],
                dst_ref=peer_ref.at[pl.ds(lo, hi - lo)],
                send_sem=send_sem.at[slot],
                recv_sem=recv_sem.at[slot],
                device_id=peer,
                device_id_type=pl.DeviceIdType.MESH,
            )

        @pl.when(i == nblk // 2 - 1)
        def _():
            pl.semaphore_wait(barrier_sem, 1)
            half_rdma(0, half, 0).start()

        @pl.when(i == nblk - 1)
        def _():
            half_rdma(half, m_per // 128, 1).start()
            half_rdma(0, half, 0).wait()
            half_rdma(half, m_per // 128, 1).wait()

            out_ref[:, :] = jnp.maximum(acc_ref[:, :], peer_ref[:, :])

    packed = pl.pallas_call(
        body,
        grid=(nblk,),
        out_shape=jax.ShapeDtypeStruct((m_per // 128, 128), x.dtype),
        in_specs=[pl.BlockSpec((bm, n_per), lambda i: (i, 0))],
        out_specs=pl.BlockSpec((m_per // 128, 128), lambda i: (0, 0)),
        scratch_shapes=[
            pltpu.VMEM((m_per // 128, 128), x.dtype),
            pltpu.VMEM((m_per // 128, 128), x.dtype),
            pltpu.SemaphoreType.DMA((2,)),
            pltpu.SemaphoreType.DMA((2,)),
        ],
        compiler_params=pltpu.CompilerParams(collective_id=0),
    )(x)
    return jnp.reshape(packed, (m_per, 1))


# device time: 8248 ns/iter; 1.0840x vs baseline; 1.0840x over previous
import jax
import jax.numpy as jnp
from jax import lax
from jax.experimental import pallas as pl
from jax.experimental.pallas import tpu as pltpu


def kernel(x):
    m_per, n_per = x.shape
    bm = 1024
    nblk = m_per // bm
    rows_per_blk = bm // 128

    def body(x_ref, out_ref, acc_ref, peer_ref, send_sem, recv_sem):
        i = pl.program_id(0)
        my_x = lax.axis_index("x")
        my_y = lax.axis_index("y")
        peer = (my_x, 1 - my_y)
        barrier_sem = pltpu.get_barrier_semaphore()

        @pl.when(i == 0)
        def _():
            pl.semaphore_signal(
                barrier_sem, inc=1, device_id=peer,
                device_id_type=pl.DeviceIdType.MESH,
            )

        s1 = x_ref[:, 0:128]
        for j in range(1, n_per // 128):
            s1 = jnp.maximum(s1, x_ref[:, j * 128:(j + 1) * 128])
        part = jnp.max(s1, axis=1)
        acc_ref[pl.ds(i * rows_per_blk, rows_per_blk), :] = jnp.reshape(
            part, (rows_per_blk, 128)
        )

        half = (nblk // 2) * rows_per_blk

        def half_rdma(lo, hi, slot):
            return pltpu.make_async_remote_copy(
                src_ref=acc_ref.at[pl.ds(lo, hi - lo)],
                dst_ref=peer_ref.at[pl.ds(lo, hi - lo)],
                send_sem=send_sem.at[slot],
                recv_sem=recv_sem.at[slot],
                device_id=peer,
                device_id_type=pl.DeviceIdType.MESH,
            )

        @pl.when(i == nblk // 2 - 1)
        def _():
            pl.semaphore_wait(barrier_sem, 1)
            half_rdma(0, half, 0).start()

        @pl.when(i == nblk - 1)
        def _():
            half_rdma(half, m_per // 128, 1).start()
            half_rdma(0, half, 0).wait()
            half_rdma(half, m_per // 128, 1).wait()

            out_ref[:, :] = jnp.maximum(acc_ref[:, :], peer_ref[:, :])

    packed = pl.pallas_call(
        body,
        grid=(nblk,),
        out_shape=jax.ShapeDtypeStruct((m_per // 128, 128), x.dtype),
        in_specs=[pl.BlockSpec((bm, n_per), lambda i: (i, 0))],
        out_specs=pl.BlockSpec((m_per // 128, 128), lambda i: (0, 0)),
        scratch_shapes=[
            pltpu.VMEM((m_per // 128, 128), x.dtype),
            pltpu.VMEM((m_per // 128, 128), x.dtype),
            pltpu.SemaphoreType.DMA((2,)),
            pltpu.SemaphoreType.DMA((2,)),
        ],
        compiler_params=pltpu.CompilerParams(collective_id=0),
    )(x)
    return jnp.reshape(packed, (m_per, 1))
